# Initial kernel scaffold; baseline (speedup 1.0000x reference)
import jax
import jax.numpy as jnp
from jax.experimental import pallas as pl


def kernel(X, edge_index, edge_weight, W1, b1, W2, b2):
    raise NotImplementedError("write your pallas kernel here")


# speedup vs baseline: 6.2404x; 6.2404x over previous
import functools

import jax
import jax.numpy as jnp
from jax import lax
from jax.experimental import pallas as pl
from jax.experimental.pallas import tpu as pltpu
from jax.experimental.pallas import tpu_sc as plsc

N_NODES = 10000
N_EDGES = 320000
IN_DIM = 128
HIDDEN = 64
H2W = 64

NC = 2
NS = 16
L = 16
EPC = N_EDGES // NC
EPT = EPC // NS
K = 80
NCHUNK = EPT // K
ACC_ROWS = 10240
ZROWS = 16


def _mesh():
    return plsc.VectorSubcoreMesh(
        core_axis_name="c", subcore_axis_name="s", num_cores=NC, num_subcores=NS
    )


def _lin1_body(x_ref, w_ref, b_ref, o_ref):
    o_ref[...] = (
        jnp.dot(x_ref[...], w_ref[...], preferred_element_type=jnp.float32)
        + b_ref[...]
    )


def _lin2_body(p_ref, w_ref, b_ref, o_ref):
    h = jnp.maximum(p_ref[0] + p_ref[1], 0.0)
    o_ref[...] = (
        jnp.dot(h, w_ref[...], preferred_element_type=jnp.float32) + b_ref[...]
    )


def _sum2_body(zp_ref, o_ref):
    o_ref[...] = zp_ref[0] + zp_ref[1]


def _spmm_body(feat_dim, h0, colr, rowr, ewr, out, col_v, row_v, ew_v, gbuf,
               zbuf, acc, sem):
    c = lax.axis_index("c")
    s = lax.axis_index("s")
    zeros16 = jnp.zeros((L,), jnp.float32)
    for r in range(ZROWS):
        for d in range(feat_dim // L):
            zbuf[r, pl.ds(d * L, L)] = zeros16

    def zloop(i, carry):
        pltpu.sync_copy(zbuf, acc.at[pl.ds(s * 640 + i * ZROWS, ZROWS)])
        return carry

    lax.fori_loop(0, 640 // ZROWS, zloop, 0)
    plsc.subcore_barrier()

    base0 = c * EPC + s * EPT

    def chunk(j, carry):
        base = base0 + j * K
        pltpu.sync_copy(colr.at[pl.ds(base, K)], col_v)
        pltpu.sync_copy(rowr.at[pl.ds(base, K)], row_v)
        pltpu.sync_copy(ewr.at[pl.ds(base, K)], ew_v)
        pltpu.async_copy(h0.at[col_v], gbuf, sem).wait()
        for g in range(K // L):
            eww = ew_v[pl.ds(g * L, L)]
            for e in range(L):
                wsc = eww[e]
                r = g * L + e
                for d in range(feat_dim // L):
                    gbuf[r, pl.ds(d * L, L)] = gbuf[r, pl.ds(d * L, L)] * wsc
        pltpu.sync_copy(gbuf, acc.at[row_v], add=True)
        return carry

    lax.fori_loop(0, NCHUNK, chunk, 0)
    plsc.subcore_barrier()

    @pl.when(s < 10)
    def _():
        pltpu.sync_copy(acc.at[pl.ds(s * 1000, 1000)], out.at[c, pl.ds(s * 1000, 1000)])


def _spmm(feat, cols, rows, ew, feat_dim):
    f = functools.partial(
        pl.kernel,
        out_type=jax.ShapeDtypeStruct((NC, N_NODES, feat_dim), jnp.float32),
        mesh=_mesh(),
        scratch_types=[
            pltpu.VMEM((K,), jnp.int32),
            pltpu.VMEM((K,), jnp.int32),
            pltpu.VMEM((K,), jnp.float32),
            pltpu.VMEM((K, feat_dim), jnp.float32),
            pltpu.VMEM((ZROWS, feat_dim), jnp.float32),
            pltpu.VMEM_SHARED((ACC_ROWS, feat_dim), jnp.float32),
            pltpu.SemaphoreType.DMA,
        ],
        compiler_params=pltpu.CompilerParams(use_tc_tiling_on_sc=False, needs_layout_passes=False),
    )(functools.partial(_spmm_body, feat_dim))
    return f(feat, cols, rows, ew)


def kernel(X, edge_index, edge_weight, W1, b1, W2, b2):
    rows = edge_index[0].astype(jnp.int32)
    cols = edge_index[1].astype(jnp.int32)
    ew = edge_weight.astype(jnp.float32)

    h0 = pl.pallas_call(
        _lin1_body,
        out_shape=jax.ShapeDtypeStruct((N_NODES, HIDDEN), jnp.float32),
    )(X, W1, b1.reshape(1, HIDDEN))

    p = _spmm(h0, cols, rows, ew, HIDDEN)

    w2w = jnp.tile(W2, (1, H2W))
    b2w = jnp.tile(b2.reshape(1, 1), (1, H2W))
    h2w = pl.pallas_call(
        _lin2_body,
        out_shape=jax.ShapeDtypeStruct((N_NODES, H2W), jnp.float32),
    )(p, w2w, b2w)

    zp = _spmm(h2w, cols, rows, ew, H2W)

    z = pl.pallas_call(
        _sum2_body,
        out_shape=jax.ShapeDtypeStruct((N_NODES, H2W), jnp.float32),
    )(zp)
    return z[:, 0]


# speedup vs baseline: 7.6395x; 1.2242x over previous
import functools

import jax
import jax.numpy as jnp
from jax import lax
from jax.experimental import pallas as pl
from jax.experimental.pallas import tpu as pltpu
from jax.experimental.pallas import tpu_sc as plsc

N_NODES = 10000
N_EDGES = 320000
IN_DIM = 128
HIDDEN = 64
H2W = 64

NC = 2
NS = 16
L = 16
EPC = N_EDGES // NC
EPT = EPC // NS
K = 80
NCHUNK = EPT // K
ACC_ROWS = 10240
ZROWS = 16


def _mesh():
    return plsc.VectorSubcoreMesh(
        core_axis_name="c", subcore_axis_name="s", num_cores=NC, num_subcores=NS
    )


def _lin1_body(x_ref, w_ref, b_ref, o_ref):
    o_ref[...] = (
        jnp.dot(x_ref[...], w_ref[...], preferred_element_type=jnp.float32)
        + b_ref[...]
    )


def _lin2_body(p_ref, w_ref, b_ref, o_ref):
    h = jnp.maximum(p_ref[0] + p_ref[1], 0.0)
    o_ref[...] = (
        jnp.dot(h, w_ref[...], preferred_element_type=jnp.float32) + b_ref[...]
    )


def _sum2_body(zp_ref, o_ref):
    o_ref[...] = zp_ref[0] + zp_ref[1]


def _spmm_body(feat_dim, h0, colr, rowr, ewr, out, col_v, row_v, ew_v, gbuf,
               zbuf, acc, sem):
    c = lax.axis_index("c")
    s = lax.axis_index("s")
    zeros16 = jnp.zeros((L,), jnp.float32)
    for r in range(ZROWS):
        for d in range(feat_dim // L):
            zbuf[r, pl.ds(d * L, L)] = zeros16

    def zloop(i, carry):
        pltpu.sync_copy(zbuf, acc.at[pl.ds(s * 640 + i * ZROWS, ZROWS)])
        return carry

    lax.fori_loop(0, 640 // ZROWS, zloop, 0)
    plsc.subcore_barrier()

    base0 = c * EPC + s * EPT

    def chunk(j, carry):
        base = base0 + j * K
        pltpu.sync_copy(colr.at[pl.ds(base, K)], col_v)
        pltpu.sync_copy(rowr.at[pl.ds(base, K)], row_v)
        pltpu.sync_copy(ewr.at[pl.ds(base, K)], ew_v)
        pltpu.async_copy(h0.at[col_v], gbuf, sem).wait()
        for g in range(K // L):
            eww = ew_v[pl.ds(g * L, L)]
            for e in range(L):
                wsc = eww[e]
                r = g * L + e
                for d in range(feat_dim // L):
                    gbuf[r, pl.ds(d * L, L)] = gbuf[r, pl.ds(d * L, L)] * wsc
        pltpu.sync_copy(gbuf, acc.at[row_v], add=True)
        return carry

    lax.fori_loop(0, NCHUNK, chunk, 0)
    plsc.subcore_barrier()

    @pl.when(s < 10)
    def _():
        pltpu.sync_copy(acc.at[pl.ds(s * 1000, 1000)], out.at[c, pl.ds(s * 1000, 1000)])


def _spmm2_body(h2, colr, rowr, ewr, out, col_v, row_v, ew_v, h2_v, gbuf,
                zbuf, acc):
    c = lax.axis_index("c")
    s = lax.axis_index("s")
    zeros16 = jnp.zeros((L,), jnp.float32)
    for r in range(ZROWS):
        zbuf[r, :] = zeros16

    def zloop(i, carry):
        pltpu.sync_copy(zbuf, acc.at[pl.ds(s * 640 + i * ZROWS, ZROWS)])
        return carry

    lax.fori_loop(0, 640 // ZROWS, zloop, 0)
    pltpu.sync_copy(h2, h2_v)
    plsc.subcore_barrier()

    base0 = c * EPC + s * EPT

    def chunk(j, carry):
        base = base0 + j * K
        pltpu.sync_copy(colr.at[pl.ds(base, K)], col_v)
        pltpu.sync_copy(rowr.at[pl.ds(base, K)], row_v)
        pltpu.sync_copy(ewr.at[pl.ds(base, K)], ew_v)
        for g in range(K // L):
            ci = col_v[pl.ds(g * L, L)]
            vals = plsc.load_gather(h2_v, [ci]) * ew_v[pl.ds(g * L, L)]
            for e in range(L):
                gbuf[g * L + e, :] = jnp.full((L,), vals[e], jnp.float32)
        pltpu.sync_copy(gbuf, acc.at[row_v], add=True)
        return carry

    lax.fori_loop(0, NCHUNK, chunk, 0)
    plsc.subcore_barrier()

    @pl.when(s < 10)
    def _():
        pltpu.sync_copy(acc.at[pl.ds(s * 1000, 1000)], out.at[c, pl.ds(s * 1000, 1000)])


def _spmm2(h2, cols, rows, ew):
    f = functools.partial(
        pl.kernel,
        out_type=jax.ShapeDtypeStruct((NC, N_NODES, L), jnp.float32),
        mesh=_mesh(),
        scratch_types=[
            pltpu.VMEM((K,), jnp.int32),
            pltpu.VMEM((K,), jnp.int32),
            pltpu.VMEM((K,), jnp.float32),
            pltpu.VMEM((N_NODES,), jnp.float32),
            pltpu.VMEM((K, L), jnp.float32),
            pltpu.VMEM((ZROWS, L), jnp.float32),
            pltpu.VMEM_SHARED((ACC_ROWS, L), jnp.float32),
        ],
        compiler_params=pltpu.CompilerParams(use_tc_tiling_on_sc=False, needs_layout_passes=False),
    )(_spmm2_body)
    return f(h2, cols, rows, ew)


def _spmm(feat, cols, rows, ew, feat_dim):
    f = functools.partial(
        pl.kernel,
        out_type=jax.ShapeDtypeStruct((NC, N_NODES, feat_dim), jnp.float32),
        mesh=_mesh(),
        scratch_types=[
            pltpu.VMEM((K,), jnp.int32),
            pltpu.VMEM((K,), jnp.int32),
            pltpu.VMEM((K,), jnp.float32),
            pltpu.VMEM((K, feat_dim), jnp.float32),
            pltpu.VMEM((ZROWS, feat_dim), jnp.float32),
            pltpu.VMEM_SHARED((ACC_ROWS, feat_dim), jnp.float32),
            pltpu.SemaphoreType.DMA,
        ],
        compiler_params=pltpu.CompilerParams(use_tc_tiling_on_sc=False, needs_layout_passes=False),
    )(functools.partial(_spmm_body, feat_dim))
    return f(feat, cols, rows, ew)


def kernel(X, edge_index, edge_weight, W1, b1, W2, b2):
    rows = edge_index[0].astype(jnp.int32)
    cols = edge_index[1].astype(jnp.int32)
    ew = edge_weight.astype(jnp.float32)

    h0 = pl.pallas_call(
        _lin1_body,
        out_shape=jax.ShapeDtypeStruct((N_NODES, HIDDEN), jnp.float32),
    )(X, W1, b1.reshape(1, HIDDEN))

    p = _spmm(h0, cols, rows, ew, HIDDEN)

    h2 = pl.pallas_call(
        _lin2_body,
        out_shape=jax.ShapeDtypeStruct((N_NODES, 1), jnp.float32),
    )(p, W2, b2.reshape(1, 1))

    zp = _spmm2(h2.reshape(N_NODES), cols, rows, ew)

    z = pl.pallas_call(
        _sum2_body,
        out_shape=jax.ShapeDtypeStruct((N_NODES, L), jnp.float32),
    )(zp)
    return z[:, 0]


# speedup vs baseline: 22.4078x; 2.9332x over previous
import functools

import jax
import jax.numpy as jnp
from jax import lax
from jax.experimental import pallas as pl
from jax.experimental.pallas import tpu as pltpu
from jax.experimental.pallas import tpu_sc as plsc

N_NODES = 10000
N_EDGES = 320000
IN_DIM = 128
HIDDEN = 64

NC = 2
NS = 16
L = 16
EPC = N_EDGES // NC
EPT = EPC // NS
K = 80
NCHUNK = EPT // K
NPAIR = NCHUNK // 2
ACC_ROWS = 10240
ZROWS = 16


def _mesh():
    return plsc.VectorSubcoreMesh(
        core_axis_name="c", subcore_axis_name="s", num_cores=NC, num_subcores=NS
    )


def _lin1_body(x_ref, w_ref, b_ref, o_ref):
    o_ref[...] = (
        jnp.dot(x_ref[...], w_ref[...], preferred_element_type=jnp.float32)
        + b_ref[...]
    )


def _lin2_body(p_ref, w_ref, b_ref, o_ref):
    h = jnp.maximum(p_ref[0] + p_ref[1], 0.0)
    o_ref[...] = (
        jnp.dot(h, w_ref[...], preferred_element_type=jnp.float32) + b_ref[...]
    )


def _sum2_body(zp_ref, o_ref):
    o_ref[...] = zp_ref[0] + zp_ref[1]


def _spmm1_body(h0, colr, rowr, ewr, out, col_v, row_v, ew_v, gbuf0, gbuf1,
                zbuf, acc, sem0, sem1):
    c = lax.axis_index("c")
    s = lax.axis_index("s")
    zeros16 = jnp.zeros((L,), jnp.float32)
    for r in range(ZROWS):
        for d in range(HIDDEN // L):
            zbuf[r, pl.ds(d * L, L)] = zeros16

    def zloop(i, carry):
        pltpu.sync_copy(zbuf, acc.at[pl.ds(s * 640 + i * ZROWS, ZROWS)])
        return carry

    lax.fori_loop(0, 640 // ZROWS, zloop, 0)

    base0 = c * EPC + s * EPT
    pltpu.sync_copy(colr.at[pl.ds(base0, EPT)], col_v)
    pltpu.sync_copy(rowr.at[pl.ds(base0, EPT)], row_v)
    pltpu.sync_copy(ewr.at[pl.ds(base0, EPT)], ew_v)
    plsc.subcore_barrier()

    def scale(gbuf, base):
        for g in range(K // L):
            eww = ew_v[pl.ds(base + g * L, L)]
            for e in range(L):
                wsc = eww[e]
                r = g * L + e
                for d in range(HIDDEN // L):
                    gbuf[r, pl.ds(d * L, L)] = gbuf[r, pl.ds(d * L, L)] * wsc

    pltpu.async_copy(h0.at[col_v.at[pl.ds(0, K)]], gbuf0, sem0)

    def pair(i, carry):
        ofsA = 2 * i * K
        ofsB = ofsA + K
        pltpu.async_copy(h0.at[col_v.at[pl.ds(ofsB, K)]], gbuf1, sem1)
        pltpu.make_async_copy(h0.at[col_v.at[pl.ds(ofsA, K)]], gbuf0, sem0).wait()
        scale(gbuf0, ofsA)
        pltpu.sync_copy(gbuf0, acc.at[row_v.at[pl.ds(ofsA, K)]], add=True)
        pltpu.async_copy(h0.at[col_v.at[pl.ds(ofsB + K, K)]], gbuf0, sem0)
        pltpu.make_async_copy(h0.at[col_v.at[pl.ds(ofsB, K)]], gbuf1, sem1).wait()
        scale(gbuf1, ofsB)
        pltpu.sync_copy(gbuf1, acc.at[row_v.at[pl.ds(ofsB, K)]], add=True)
        return carry

    lax.fori_loop(0, NPAIR, pair, 0)

    ofsZ = NPAIR * 2 * K
    pltpu.make_async_copy(h0.at[col_v.at[pl.ds(ofsZ, K)]], gbuf0, sem0).wait()
    scale(gbuf0, ofsZ)
    pltpu.sync_copy(gbuf0, acc.at[row_v.at[pl.ds(ofsZ, K)]], add=True)
    plsc.subcore_barrier()

    @pl.when(s < 10)
    def _():
        pltpu.sync_copy(acc.at[pl.ds(s * 1000, 1000)], out.at[c, pl.ds(s * 1000, 1000)])


def _spmm1(feat, cols, rows, ew):
    f = functools.partial(
        pl.kernel,
        out_type=jax.ShapeDtypeStruct((NC, N_NODES, HIDDEN), jnp.float32),
        mesh=_mesh(),
        scratch_types=[
            pltpu.VMEM((EPT,), jnp.int32),
            pltpu.VMEM((EPT,), jnp.int32),
            pltpu.VMEM((EPT,), jnp.float32),
            pltpu.VMEM((K, HIDDEN), jnp.float32),
            pltpu.VMEM((K, HIDDEN), jnp.float32),
            pltpu.VMEM((ZROWS, HIDDEN), jnp.float32),
            pltpu.VMEM_SHARED((ACC_ROWS, HIDDEN), jnp.float32),
            pltpu.SemaphoreType.DMA,
            pltpu.SemaphoreType.DMA,
        ],
        compiler_params=pltpu.CompilerParams(use_tc_tiling_on_sc=False, needs_layout_passes=False),
    )(_spmm1_body)
    return f(feat, cols, rows, ew)


def _spmm2_body(h2, colr, rowr, ewr, out, col_v, row_v, ew_v, h2_v, gbuf0,
                gbuf1, zbuf, acc, sem0, sem1):
    c = lax.axis_index("c")
    s = lax.axis_index("s")
    zeros16 = jnp.zeros((L,), jnp.float32)
    for r in range(ZROWS):
        zbuf[r, :] = zeros16

    def zloop(i, carry):
        pltpu.sync_copy(zbuf, acc.at[pl.ds(s * 640 + i * ZROWS, ZROWS)])
        return carry

    lax.fori_loop(0, 640 // ZROWS, zloop, 0)
    pltpu.sync_copy(h2, h2_v)
    base0 = c * EPC + s * EPT
    pltpu.sync_copy(colr.at[pl.ds(base0, EPT)], col_v)
    pltpu.sync_copy(rowr.at[pl.ds(base0, EPT)], row_v)
    pltpu.sync_copy(ewr.at[pl.ds(base0, EPT)], ew_v)
    plsc.subcore_barrier()

    def splat(gbuf, base):
        for g in range(K // L):
            ci = col_v[pl.ds(base + g * L, L)]
            vals = plsc.load_gather(h2_v, [ci]) * ew_v[pl.ds(base + g * L, L)]
            for e in range(L):
                gbuf[g * L + e, :] = jnp.full((L,), vals[e], jnp.float32)

    splat(gbuf0, 0)
    pltpu.async_copy(gbuf0, acc.at[row_v.at[pl.ds(0, K)]], sem0, add=True)

    def pair(i, carry):
        ofsA = 2 * i * K
        ofsB = ofsA + K
        splat(gbuf1, ofsB)
        pltpu.async_copy(gbuf1, acc.at[row_v.at[pl.ds(ofsB, K)]], sem1, add=True)
        pltpu.make_async_copy(gbuf0, acc.at[row_v.at[pl.ds(ofsA, K)]], sem0).wait()
        splat(gbuf0, ofsB + K)
        pltpu.async_copy(gbuf0, acc.at[row_v.at[pl.ds(ofsB + K, K)]], sem0, add=True)
        pltpu.make_async_copy(gbuf1, acc.at[row_v.at[pl.ds(ofsB, K)]], sem1).wait()
        return carry

    lax.fori_loop(0, NPAIR, pair, 0)
    ofsZ = NPAIR * 2 * K
    pltpu.make_async_copy(gbuf0, acc.at[row_v.at[pl.ds(ofsZ, K)]], sem0).wait()
    plsc.subcore_barrier()

    @pl.when(s < 10)
    def _():
        pltpu.sync_copy(acc.at[pl.ds(s * 1000, 1000)], out.at[c, pl.ds(s * 1000, 1000)])


def _spmm2(h2, cols, rows, ew):
    f = functools.partial(
        pl.kernel,
        out_type=jax.ShapeDtypeStruct((NC, N_NODES, L), jnp.float32),
        mesh=_mesh(),
        scratch_types=[
            pltpu.VMEM((EPT,), jnp.int32),
            pltpu.VMEM((EPT,), jnp.int32),
            pltpu.VMEM((EPT,), jnp.float32),
            pltpu.VMEM((N_NODES,), jnp.float32),
            pltpu.VMEM((K, L), jnp.float32),
            pltpu.VMEM((K, L), jnp.float32),
            pltpu.VMEM((ZROWS, L), jnp.float32),
            pltpu.VMEM_SHARED((ACC_ROWS, L), jnp.float32),
            pltpu.SemaphoreType.DMA,
            pltpu.SemaphoreType.DMA,
        ],
        compiler_params=pltpu.CompilerParams(use_tc_tiling_on_sc=False, needs_layout_passes=False),
    )(_spmm2_body)
    return f(h2, cols, rows, ew)


def kernel(X, edge_index, edge_weight, W1, b1, W2, b2):
    rows = edge_index[0].astype(jnp.int32)
    cols = edge_index[1].astype(jnp.int32)
    ew = edge_weight.astype(jnp.float32)

    h0 = pl.pallas_call(
        _lin1_body,
        out_shape=jax.ShapeDtypeStruct((N_NODES, HIDDEN), jnp.float32),
    )(X, W1, b1.reshape(1, HIDDEN))

    p = _spmm1(h0, cols, rows, ew)

    h2 = pl.pallas_call(
        _lin2_body,
        out_shape=jax.ShapeDtypeStruct((N_NODES, 1), jnp.float32),
    )(p, W2, b2.reshape(1, 1))

    zp = _spmm2(h2.reshape(N_NODES), cols, rows, ew)

    z = pl.pallas_call(
        _sum2_body,
        out_shape=jax.ShapeDtypeStruct((N_NODES, L), jnp.float32),
    )(zp)
    return z[:, 0]
